# baseline (speedup 1.0000x reference)
import functools

import jax
import jax.numpy as jnp
from jax import lax
from jax.experimental import pallas as pl
from jax.experimental.pallas import tpu as pltpu
from jax.experimental.pallas import tpu_sc as plsc

HD = 128
N = 10000
E = 320000
LANES = 16

NB = 1000
BE = 2000
KMIC = 40
NMICRO = 5
KMAC = KMIC * NMICRO
KMIC2 = 80
NC = 2
NS = 16
NW = NC * NS
EPW = E // NW
NMAC = EPW // KMAC
NMACB = EPW // (KMIC2 * NMICRO)
ROWS_PT = 624
ROWS_LAST = N - (NS - 1) * ROWS_PT


def _silu(x):
    return x * lax.logistic(x)


def _prep_body(an, emb, w1u, w1v, b1, w2u, w2v, b2,
               h_o, pu1_o, pv1_o, pu2_o, pv2_o):
    iota = lax.broadcasted_iota(jnp.int32, (1, HD), 1)
    oh = (an[...] == iota).astype(jnp.float32)
    h = jnp.dot(oh, emb[...], preferred_element_type=jnp.float32)
    h_o[...] = h
    pu1_o[...] = jnp.dot(h, w1u[...], preferred_element_type=jnp.float32) + b1[...]
    pv1_o[...] = jnp.dot(h, w1v[...], preferred_element_type=jnp.float32)
    pu2_o[...] = jnp.dot(h, w2u[...], preferred_element_type=jnp.float32) + b2[...]
    pv2_o[...] = jnp.dot(h, w2v[...], preferred_element_type=jnp.float32)


def _tc_prep(atom2d, emb_pad, w1u, w1v, b1, w2u, w2v, b2):
    row = pl.BlockSpec((NB, HD), lambda i: (i, 0))
    wsp = pl.BlockSpec((HD, HD), lambda i: (0, 0))
    bsp = pl.BlockSpec((1, HD), lambda i: (0, 0))
    return pl.pallas_call(
        _prep_body,
        grid=(N // NB,),
        in_specs=[pl.BlockSpec((NB, 1), lambda i: (i, 0)),
                  wsp, wsp, wsp, bsp, wsp, wsp, bsp],
        out_specs=[row] * 5,
        out_shape=[jax.ShapeDtypeStruct((N, HD), jnp.float32)] * 5,
    )(atom2d, emb_pad, w1u, w1v, b1, w2u, w2v, b2)


def _emlp_body(ga, gb, dis, w1d, w2, b2, w3, b3, m_o):
    x = _silu(ga[...] + gb[...] + dis[...] * w1d[...])
    x = _silu(jnp.dot(x, w2[...], preferred_element_type=jnp.float32) + b2[...])
    m_o[...] = jnp.dot(x, w3[...], preferred_element_type=jnp.float32) + b3[...]


def _tc_emlp(ga, gb, dis2d, w1d, w2, b2, w3, b3):
    row = pl.BlockSpec((BE, HD), lambda i: (i, 0))
    wsp = pl.BlockSpec((HD, HD), lambda i: (0, 0))
    bsp = pl.BlockSpec((1, HD), lambda i: (0, 0))
    return pl.pallas_call(
        _emlp_body,
        grid=(E // BE,),
        in_specs=[row, row, pl.BlockSpec((BE, 1), lambda i: (i, 0)),
                  bsp, wsp, bsp, wsp, bsp],
        out_specs=row,
        out_shape=jax.ShapeDtypeStruct((E, HD), jnp.float32),
    )(ga, gb, dis2d, w1d, w2, b2, w3, b3)


def _upd_common(h, s1, s2, uh, ua, ub, b1, w2, b2):
    a1 = s1[0] + s1[1]
    a2 = s2[0] + s2[1]
    x = _silu(jnp.dot(h[...], uh[...], preferred_element_type=jnp.float32)
              + jnp.dot(a1, ua[...], preferred_element_type=jnp.float32)
              + jnp.dot(a2, ub[...], preferred_element_type=jnp.float32)
              + b1[...])
    x = jnp.dot(x, w2[...], preferred_element_type=jnp.float32) + b2[...]
    return h[...] + x


def _upd1_body(h, s1, s2, uh, ua, ub, b1, w2, b2,
               qw1u, qw1v, qb1, qw2u, qw2v, qb2,
               h_o, pu1_o, pv1_o, pu2_o, pv2_o):
    hn = _upd_common(h, s1, s2, uh, ua, ub, b1, w2, b2)
    h_o[...] = hn
    pu1_o[...] = jnp.dot(hn, qw1u[...], preferred_element_type=jnp.float32) + qb1[...]
    pv1_o[...] = jnp.dot(hn, qw1v[...], preferred_element_type=jnp.float32)
    pu2_o[...] = jnp.dot(hn, qw2u[...], preferred_element_type=jnp.float32) + qb2[...]
    pv2_o[...] = jnp.dot(hn, qw2v[...], preferred_element_type=jnp.float32)


def _tc_upd1(h, s1, s2, uh, ua, ub, b1, w2, b2,
             qw1u, qw1v, qb1, qw2u, qw2v, qb2):
    row = pl.BlockSpec((NB, HD), lambda i: (i, 0))
    ssp = pl.BlockSpec((NC, NB, HD), lambda i: (0, i, 0))
    wsp = pl.BlockSpec((HD, HD), lambda i: (0, 0))
    bsp = pl.BlockSpec((1, HD), lambda i: (0, 0))
    return pl.pallas_call(
        _upd1_body,
        grid=(N // NB,),
        in_specs=[row, ssp, ssp, wsp, wsp, wsp, bsp, wsp, bsp,
                  wsp, wsp, bsp, wsp, wsp, bsp],
        out_specs=[row] * 5,
        out_shape=[jax.ShapeDtypeStruct((N, HD), jnp.float32)] * 5,
    )(h, s1, s2, uh, ua, ub, b1, w2, b2, qw1u, qw1v, qb1, qw2u, qw2v, qb2)


def _upd2_body(h, s1, s2, uh, ua, ub, b1, w2, b2, wout, bout, d_o):
    hn = _upd_common(h, s1, s2, uh, ua, ub, b1, w2, b2)
    d_o[...] = jnp.dot(hn, wout[...], preferred_element_type=jnp.float32) + bout[...]


def _tc_upd2(h, s1, s2, uh, ua, ub, b1, w2, b2, wout_pad, bout_pad):
    row = pl.BlockSpec((NB, HD), lambda i: (i, 0))
    ssp = pl.BlockSpec((NC, NB, HD), lambda i: (0, i, 0))
    wsp = pl.BlockSpec((HD, HD), lambda i: (0, 0))
    bsp = pl.BlockSpec((1, HD), lambda i: (0, 0))
    return pl.pallas_call(
        _upd2_body,
        grid=(N // NB,),
        in_specs=[row, ssp, ssp, wsp, wsp, wsp, bsp, wsp, bsp,
                  pl.BlockSpec((HD, 8), lambda i: (0, 0)),
                  pl.BlockSpec((1, 8), lambda i: (0, 0))],
        out_specs=pl.BlockSpec((NB, 8), lambda i: (i, 0)),
        out_shape=jax.ShapeDtypeStruct((N, 8), jnp.float32),
    )(h, s1, s2, uh, ua, ub, b1, w2, b2, wout_pad, bout_pad)


@functools.lru_cache(maxsize=None)
def _sc_gather_fn():
    mesh = plsc.VectorSubcoreMesh(core_axis_name="c", subcore_axis_name="s")

    @functools.partial(
        pl.kernel, mesh=mesh,
        out_type=[jax.ShapeDtypeStruct((E, HD), jnp.float32),
                  jax.ShapeDtypeStruct((E, HD), jnp.float32)],
        scratch_types=[
            pltpu.VMEM((2 * KMAC,), jnp.int32),
            pltpu.VMEM((2 * KMAC,), jnp.int32),
            pltpu.VMEM((KMAC, HD), jnp.float32),
            pltpu.VMEM((KMAC, HD), jnp.float32),
            pltpu.VMEM((KMAC, HD), jnp.float32),
            pltpu.VMEM((KMAC, HD), jnp.float32),
            pltpu.SemaphoreType.DMA,
            pltpu.SemaphoreType.DMA,
            pltpu.SemaphoreType.DMA,
        ])
    def _gather(pu_hbm, pv_hbm, iu_hbm, iv_hbm, outa_hbm, outb_hbm,
                iu_v, iv_v, a0_v, b0_v, a1_v, b1_v, sem0, sem1, sem_o):
        wid = lax.axis_index("s") * NC + lax.axis_index("c")
        ebase = wid * EPW
        slots = ((a0_v, b0_v, sem0), (a1_v, b1_v, sem1))

        def pair(i, carry):
            e0 = ebase + i * (2 * KMAC)
            pltpu.sync_copy(iu_hbm.at[pl.ds(e0, 2 * KMAC)], iu_v)
            pltpu.sync_copy(iv_hbm.at[pl.ds(e0, 2 * KMAC)], iv_v)
            gcps = []
            for sl, (a_v, b_v, sem) in enumerate(slots):
                cps = []
                for j in range(NMICRO):
                    src = pl.ds(sl * KMAC + j * KMIC, KMIC)
                    dst = pl.ds(j * KMIC, KMIC)
                    cps.append(pltpu.async_copy(pu_hbm.at[iu_v.at[src]],
                                                a_v.at[dst], sem))
                    cps.append(pltpu.async_copy(pv_hbm.at[iv_v.at[src]],
                                                b_v.at[dst], sem))
                gcps.append(cps)
            ocps = []
            for sl, (a_v, b_v, sem) in enumerate(slots):
                for cp in gcps[sl]:
                    cp.wait()
                osl = pl.ds(e0 + sl * KMAC, KMAC)
                ocps.append(pltpu.async_copy(a_v, outa_hbm.at[osl], sem_o))
                ocps.append(pltpu.async_copy(b_v, outb_hbm.at[osl], sem_o))
            for cp in ocps:
                cp.wait()
            return carry

        lax.fori_loop(0, NMAC // 2, pair, 0)

    return _gather


def _sc_gather(pu, pv, iu, iv):
    return _sc_gather_fn()(pu, pv, iu, iv)


@functools.lru_cache(maxsize=None)
def _sc_scatter_fn():
    mesh = plsc.VectorSubcoreMesh(core_axis_name="c", subcore_axis_name="s")

    @functools.partial(
        pl.kernel, mesh=mesh,
        out_type=jax.ShapeDtypeStruct((NC, N, HD), jnp.float32),
        scratch_types=[
            pltpu.VMEM((NMICRO, KMIC2), jnp.int32),
            pltpu.VMEM((KMIC2, HD), jnp.float32),
            pltpu.VMEM((KMIC2, HD), jnp.float32),
            pltpu.VMEM_SHARED((N, HD), jnp.float32),
            pltpu.SemaphoreType.DMA,
            pltpu.SemaphoreType.DMA,
        ])
    def _scatter(m_hbm, iv_hbm, z_hbm, out_hbm, iv_v, m0_v, m1_v, s_sh,
                 sem_m, sem_s):
        c = lax.axis_index("c")
        s = lax.axis_index("s")
        wid = s * NC + c
        r0 = s * ROWS_PT

        @pl.when(s < NS - 1)
        def _():
            pltpu.sync_copy(z_hbm.at[pl.ds(r0, ROWS_PT)], s_sh.at[pl.ds(r0, ROWS_PT)])

        @pl.when(s == NS - 1)
        def _():
            pltpu.sync_copy(z_hbm.at[pl.ds(r0, ROWS_LAST)], s_sh.at[pl.ds(r0, ROWS_LAST)])

        plsc.subcore_barrier()
        ebase = wid * EPW

        bufs = (m0_v, m1_v)

        def mac(ci, carry):
            mid = wid * NMACB + ci
            pltpu.sync_copy(iv_hbm.at[mid], iv_v)
            e0 = ebase + ci * KMIC2 * NMICRO

            def mcp(j):
                return pltpu.async_copy(
                    m_hbm.at[pl.ds(e0 + j * KMIC2, KMIC2)],
                    bufs[j % 2], sem_m)

            def scp(j):
                return pltpu.async_copy(bufs[j % 2], s_sh.at[iv_v.at[j]],
                                        sem_s, add=True)

            cpm = {0: mcp(0), 1: mcp(1)}
            cps = {}
            for j in range(NMICRO):
                cpm[j].wait()
                cps[j] = scp(j)
                if j + 2 < NMICRO:
                    cps[j].wait()
                    cpm[j + 2] = mcp(j + 2)
            for j in range(max(0, NMICRO - 2), NMICRO):
                cps[j].wait()
            return carry

        lax.fori_loop(0, NMACB, mac, 0)
        plsc.subcore_barrier()

        @pl.when(s < NS - 1)
        def _():
            pltpu.sync_copy(s_sh.at[pl.ds(r0, ROWS_PT)],
                            out_hbm.at[c].at[pl.ds(r0, ROWS_PT)])

        @pl.when(s == NS - 1)
        def _():
            pltpu.sync_copy(s_sh.at[pl.ds(r0, ROWS_LAST)],
                            out_hbm.at[c].at[pl.ds(r0, ROWS_LAST)])

    return _scatter


def _sc_scatter(m, iv, z):
    return _sc_scatter_fn()(m, iv, z)


def _split_edge_w(ep):
    w1 = ep['W1']
    return (w1[:HD], w1[HD:2 * HD], w1[2 * HD:2 * HD + 1],
            ep['b1'].reshape(1, HD), ep['W2'], ep['b2'].reshape(1, HD),
            ep['W3'], ep['b3'].reshape(1, HD))


def _split_upd_w(up):
    w1 = up['W1']
    return (w1[:HD], w1[HD:2 * HD], w1[2 * HD:], up['b1'].reshape(1, HD),
            up['W2'], up['b2'].reshape(1, HD))


def kernel(atom_num, dis1, dis2, id1u, id1v, id2u, id2v, params):
    p = params
    atom2d = atom_num.astype(jnp.int32).reshape(N, 1)
    dis1_2d = dis1.reshape(E, 1)
    dis2_2d = dis2.reshape(E, 1)
    i1u = id1u.astype(jnp.int32)
    i1v = id1v.astype(jnp.int32)
    i2u = id2u.astype(jnp.int32)
    i2v = id2v.astype(jnp.int32)
    i1v3 = i1v.reshape(-1, NMICRO, KMIC2)
    i2v3 = i2v.reshape(-1, NMICRO, KMIC2)
    z = jnp.zeros((N, HD), jnp.float32)

    emb = p['atom_emb']
    emb_pad = jnp.pad(emb, ((0, HD - emb.shape[0]), (0, 0)))

    e1u, e1v, e1d, e1b1, e1w2, e1b2, e1w3, e1b3 = _split_edge_w(p['edge1'])
    e2u, e2v, e2d, e2b1, e2w2, e2b2, e2w3, e2b3 = _split_edge_w(p['edge2'])
    f1u, f1v, f1d, f1b1, f1w2, f1b2, f1w3, f1b3 = _split_edge_w(p['uedge1'])
    f2u, f2v, f2d, f2b1, f2w2, f2b2, f2w3, f2b3 = _split_edge_w(p['uedge2'])
    u1h, u1a, u1b, u1b1, u1w2, u1b2 = _split_upd_w(p['upd1'])
    u2h, u2a, u2b, u2b1, u2w2, u2b2 = _split_upd_w(p['upd2'])

    wout_pad = jnp.pad(p['Wout'], ((0, 0), (0, 8 - p['Wout'].shape[1])))
    bout_pad = jnp.pad(p['bout'], (0, 8 - p['bout'].shape[0])).reshape(1, 8)

    h, pu1, pv1, pu2, pv2 = _tc_prep(atom2d, emb_pad, e1u, e1v, e1b1,
                                     e2u, e2v, e2b1)
    g1a, g1b = _sc_gather(pu1, pv1, i1u, i1v)
    g2a, g2b = _sc_gather(pu2, pv2, i2u, i2v)
    m1 = _tc_emlp(g1a, g1b, dis1_2d, e1d, e1w2, e1b2, e1w3, e1b3)
    m2 = _tc_emlp(g2a, g2b, dis2_2d, e2d, e2w2, e2b2, e2w3, e2b3)
    s1 = _sc_scatter(m1, i1v3, z)
    s2 = _sc_scatter(m2, i2v3, z)
    h1, qu1, qv1, qu2, qv2 = _tc_upd1(h, s1, s2, u1h, u1a, u1b, u1b1, u1w2,
                                      u1b2, f1u, f1v, f1b1, f2u, f2v, f2b1)

    g1a, g1b = _sc_gather(qu1, qv1, i1u, i1v)
    g2a, g2b = _sc_gather(qu2, qv2, i2u, i2v)
    m1 = _tc_emlp(g1a, g1b, dis1_2d, f1d, f1w2, f1b2, f1w3, f1b3)
    m2 = _tc_emlp(g2a, g2b, dis2_2d, f2d, f2w2, f2b2, f2w3, f2b3)
    s1 = _sc_scatter(m1, i1v3, z)
    s2 = _sc_scatter(m2, i2v3, z)
    delta8 = _tc_upd2(h1, s1, s2, u2h, u2a, u2b, u2b1, u2w2, u2b2,
                      wout_pad, bout_pad)
    return delta8[:, :3]


# speedup vs baseline: 1.1182x; 1.1182x over previous
import functools

import jax
import jax.numpy as jnp
from jax import lax
from jax.experimental import pallas as pl
from jax.experimental.pallas import tpu as pltpu
from jax.experimental.pallas import tpu_sc as plsc

HD = 128
N = 10000
E = 320000
LANES = 16

NB = 1000
BE = 2000
KMIC = 40
NMICRO = 5
KMAC = KMIC * NMICRO
KMIC2 = 80
NC = 2
NS = 16
NW = NC * NS
EPW = E // NW
NMAC = EPW // KMAC
NMACB = EPW // (KMIC2 * NMICRO)
ROWS_PT = 624
ROWS_LAST = N - (NS - 1) * ROWS_PT


def _silu(x):
    return x * lax.logistic(x)


def _prep_body(an, emb, w1u, w1v, b1, w2u, w2v, b2,
               h_o, pu1_o, pv1_o, pu2_o, pv2_o):
    iota = lax.broadcasted_iota(jnp.int32, (1, HD), 1)
    oh = (an[...] == iota).astype(jnp.float32)
    h = jnp.dot(oh, emb[...], preferred_element_type=jnp.float32)
    h_o[...] = h
    pu1_o[...] = jnp.dot(h, w1u[...], preferred_element_type=jnp.float32) + b1[...]
    pv1_o[...] = jnp.dot(h, w1v[...], preferred_element_type=jnp.float32)
    pu2_o[...] = jnp.dot(h, w2u[...], preferred_element_type=jnp.float32) + b2[...]
    pv2_o[...] = jnp.dot(h, w2v[...], preferred_element_type=jnp.float32)


def _tc_prep(atom2d, emb_pad, w1u, w1v, b1, w2u, w2v, b2):
    row = pl.BlockSpec((NB, HD), lambda i: (i, 0))
    wsp = pl.BlockSpec((HD, HD), lambda i: (0, 0))
    bsp = pl.BlockSpec((1, HD), lambda i: (0, 0))
    return pl.pallas_call(
        _prep_body,
        grid=(N // NB,),
        in_specs=[pl.BlockSpec((NB, 1), lambda i: (i, 0)),
                  wsp, wsp, wsp, bsp, wsp, wsp, bsp],
        out_specs=[row] * 5,
        out_shape=[jax.ShapeDtypeStruct((N, HD), jnp.float32)] * 5,
    )(atom2d, emb_pad, w1u, w1v, b1, w2u, w2v, b2)


def _emlp_body(g, dis, w1d, w2, b2, w3, b3, m_o):
    x = _silu(g[...] + dis[...] * w1d[...])
    x = _silu(jnp.dot(x, w2[...], preferred_element_type=jnp.float32) + b2[...])
    m_o[...] = jnp.dot(x, w3[...], preferred_element_type=jnp.float32) + b3[...]


def _tc_emlp(g, dis2d, w1d, w2, b2, w3, b3):
    row = pl.BlockSpec((BE, HD), lambda i: (i, 0))
    wsp = pl.BlockSpec((HD, HD), lambda i: (0, 0))
    bsp = pl.BlockSpec((1, HD), lambda i: (0, 0))
    return pl.pallas_call(
        _emlp_body,
        grid=(E // BE,),
        in_specs=[row, pl.BlockSpec((BE, 1), lambda i: (i, 0)),
                  bsp, wsp, bsp, wsp, bsp],
        out_specs=row,
        out_shape=jax.ShapeDtypeStruct((E, HD), jnp.float32),
    )(g, dis2d, w1d, w2, b2, w3, b3)


def _upd_common(h, s1, s2, uh, ua, ub, b1, w2, b2):
    a1 = s1[0] + s1[1]
    a2 = s2[0] + s2[1]
    x = _silu(jnp.dot(h[...], uh[...], preferred_element_type=jnp.float32)
              + jnp.dot(a1, ua[...], preferred_element_type=jnp.float32)
              + jnp.dot(a2, ub[...], preferred_element_type=jnp.float32)
              + b1[...])
    x = jnp.dot(x, w2[...], preferred_element_type=jnp.float32) + b2[...]
    return h[...] + x


def _upd1_body(h, s1, s2, uh, ua, ub, b1, w2, b2,
               qw1u, qw1v, qb1, qw2u, qw2v, qb2,
               h_o, pu1_o, pv1_o, pu2_o, pv2_o):
    hn = _upd_common(h, s1, s2, uh, ua, ub, b1, w2, b2)
    h_o[...] = hn
    pu1_o[...] = jnp.dot(hn, qw1u[...], preferred_element_type=jnp.float32) + qb1[...]
    pv1_o[...] = jnp.dot(hn, qw1v[...], preferred_element_type=jnp.float32)
    pu2_o[...] = jnp.dot(hn, qw2u[...], preferred_element_type=jnp.float32) + qb2[...]
    pv2_o[...] = jnp.dot(hn, qw2v[...], preferred_element_type=jnp.float32)


def _tc_upd1(h, s1, s2, uh, ua, ub, b1, w2, b2,
             qw1u, qw1v, qb1, qw2u, qw2v, qb2):
    row = pl.BlockSpec((NB, HD), lambda i: (i, 0))
    ssp = pl.BlockSpec((NC, NB, HD), lambda i: (0, i, 0))
    wsp = pl.BlockSpec((HD, HD), lambda i: (0, 0))
    bsp = pl.BlockSpec((1, HD), lambda i: (0, 0))
    return pl.pallas_call(
        _upd1_body,
        grid=(N // NB,),
        in_specs=[row, ssp, ssp, wsp, wsp, wsp, bsp, wsp, bsp,
                  wsp, wsp, bsp, wsp, wsp, bsp],
        out_specs=[row] * 5,
        out_shape=[jax.ShapeDtypeStruct((N, HD), jnp.float32)] * 5,
    )(h, s1, s2, uh, ua, ub, b1, w2, b2, qw1u, qw1v, qb1, qw2u, qw2v, qb2)


def _upd2_body(h, s1, s2, uh, ua, ub, b1, w2, b2, wout, bout, d_o):
    hn = _upd_common(h, s1, s2, uh, ua, ub, b1, w2, b2)
    d_o[...] = jnp.dot(hn, wout[...], preferred_element_type=jnp.float32) + bout[...]


def _tc_upd2(h, s1, s2, uh, ua, ub, b1, w2, b2, wout_pad, bout_pad):
    row = pl.BlockSpec((NB, HD), lambda i: (i, 0))
    ssp = pl.BlockSpec((NC, NB, HD), lambda i: (0, i, 0))
    wsp = pl.BlockSpec((HD, HD), lambda i: (0, 0))
    bsp = pl.BlockSpec((1, HD), lambda i: (0, 0))
    return pl.pallas_call(
        _upd2_body,
        grid=(N // NB,),
        in_specs=[row, ssp, ssp, wsp, wsp, wsp, bsp, wsp, bsp,
                  pl.BlockSpec((HD, 8), lambda i: (0, 0)),
                  pl.BlockSpec((1, 8), lambda i: (0, 0))],
        out_specs=pl.BlockSpec((NB, 8), lambda i: (i, 0)),
        out_shape=jax.ShapeDtypeStruct((N, 8), jnp.float32),
    )(h, s1, s2, uh, ua, ub, b1, w2, b2, wout_pad, bout_pad)


@functools.lru_cache(maxsize=None)
def _sc_gather_fn():
    mesh = plsc.VectorSubcoreMesh(core_axis_name="c", subcore_axis_name="s")

    @functools.partial(
        pl.kernel, mesh=mesh,
        out_type=jax.ShapeDtypeStruct((E, HD), jnp.float32),
        scratch_types=[
            pltpu.VMEM((2 * KMAC,), jnp.int32),
            pltpu.VMEM((2 * KMAC,), jnp.int32),
            pltpu.VMEM((KMAC, HD), jnp.float32),
            pltpu.VMEM((KMAC, HD), jnp.float32),
            pltpu.VMEM((KMAC, HD), jnp.float32),
            pltpu.VMEM((KMAC, HD), jnp.float32),
            pltpu.SemaphoreType.DMA,
            pltpu.SemaphoreType.DMA,
            pltpu.SemaphoreType.DMA,
        ])
    def _gather(pu_hbm, pv_hbm, iu_hbm, iv_hbm, out_hbm,
                iu_v, iv_v, a0_v, b0_v, a1_v, b1_v, sem0, sem1, sem_o):
        wid = lax.axis_index("s") * NC + lax.axis_index("c")
        ebase = wid * EPW
        slots = ((a0_v, b0_v, sem0), (a1_v, b1_v, sem1))

        def fire(sl, e0):
            a_v, b_v, sem = slots[sl]
            cps = []
            for j in range(NMICRO):
                src = pl.ds(sl * KMAC + j * KMIC, KMIC)
                dst = pl.ds(j * KMIC, KMIC)
                cps.append(pltpu.async_copy(pu_hbm.at[iu_v.at[src]],
                                            a_v.at[dst], sem))
                cps.append(pltpu.async_copy(pv_hbm.at[iv_v.at[src]],
                                            b_v.at[dst], sem))
            return cps

        def add_and_out(sl, e0):
            a_v, b_v, _ = slots[sl]

            def add_row(e, c2):
                for j in range(HD // LANES):
                    lsl = pl.ds(j * LANES, LANES)
                    a_v[e, lsl] = a_v[e, lsl] + b_v[e, lsl]
                return c2

            lax.fori_loop(0, KMAC, add_row, 0)
            return pltpu.async_copy(a_v, out_hbm.at[pl.ds(e0 + sl * KMAC, KMAC)],
                                    sem_o)

        def pair(i, carry):
            e0 = ebase + i * (2 * KMAC)
            pltpu.sync_copy(iu_hbm.at[pl.ds(e0, 2 * KMAC)], iu_v)
            pltpu.sync_copy(iv_hbm.at[pl.ds(e0, 2 * KMAC)], iv_v)
            cps0 = fire(0, e0)
            cps1 = fire(1, e0)
            for cp in cps0:
                cp.wait()
            o0 = add_and_out(0, e0)
            for cp in cps1:
                cp.wait()
            o1 = add_and_out(1, e0)
            o0.wait()
            o1.wait()
            return carry

        lax.fori_loop(0, NMAC // 2, pair, 0)

    return _gather


def _sc_gather(pu, pv, iu, iv):
    return _sc_gather_fn()(pu, pv, iu, iv)


@functools.lru_cache(maxsize=None)
def _sc_scatter_fn():
    mesh = plsc.VectorSubcoreMesh(core_axis_name="c", subcore_axis_name="s")

    @functools.partial(
        pl.kernel, mesh=mesh,
        out_type=jax.ShapeDtypeStruct((NC, N, HD), jnp.float32),
        scratch_types=[
            pltpu.VMEM((NMICRO, KMIC2), jnp.int32),
            pltpu.VMEM((KMIC2, HD), jnp.float32),
            pltpu.VMEM((KMIC2, HD), jnp.float32),
            pltpu.VMEM_SHARED((N, HD), jnp.float32),
            pltpu.SemaphoreType.DMA,
            pltpu.SemaphoreType.DMA,
        ])
    def _scatter(m_hbm, iv_hbm, z_hbm, out_hbm, iv_v, m0_v, m1_v, s_sh,
                 sem_m, sem_s):
        c = lax.axis_index("c")
        s = lax.axis_index("s")
        wid = s * NC + c
        r0 = s * ROWS_PT

        @pl.when(s < NS - 1)
        def _():
            pltpu.sync_copy(z_hbm.at[pl.ds(r0, ROWS_PT)], s_sh.at[pl.ds(r0, ROWS_PT)])

        @pl.when(s == NS - 1)
        def _():
            pltpu.sync_copy(z_hbm.at[pl.ds(r0, ROWS_LAST)], s_sh.at[pl.ds(r0, ROWS_LAST)])

        plsc.subcore_barrier()
        ebase = wid * EPW

        bufs = (m0_v, m1_v)

        def mac(ci, carry):
            mid = wid * NMACB + ci
            pltpu.sync_copy(iv_hbm.at[mid], iv_v)
            e0 = ebase + ci * KMIC2 * NMICRO

            def mcp(j):
                return pltpu.async_copy(
                    m_hbm.at[pl.ds(e0 + j * KMIC2, KMIC2)],
                    bufs[j % 2], sem_m)

            def scp(j):
                return pltpu.async_copy(bufs[j % 2], s_sh.at[iv_v.at[j]],
                                        sem_s, add=True)

            cpm = {0: mcp(0), 1: mcp(1)}
            cps = {}
            for j in range(NMICRO):
                cpm[j].wait()
                cps[j] = scp(j)
                if j + 2 < NMICRO:
                    cps[j].wait()
                    cpm[j + 2] = mcp(j + 2)
            for j in range(max(0, NMICRO - 2), NMICRO):
                cps[j].wait()
            return carry

        lax.fori_loop(0, NMACB, mac, 0)
        plsc.subcore_barrier()

        @pl.when(s < NS - 1)
        def _():
            pltpu.sync_copy(s_sh.at[pl.ds(r0, ROWS_PT)],
                            out_hbm.at[c].at[pl.ds(r0, ROWS_PT)])

        @pl.when(s == NS - 1)
        def _():
            pltpu.sync_copy(s_sh.at[pl.ds(r0, ROWS_LAST)],
                            out_hbm.at[c].at[pl.ds(r0, ROWS_LAST)])

    return _scatter


def _sc_scatter(m, iv, z):
    return _sc_scatter_fn()(m, iv, z)


def _split_edge_w(ep):
    w1 = ep['W1']
    return (w1[:HD], w1[HD:2 * HD], w1[2 * HD:2 * HD + 1],
            ep['b1'].reshape(1, HD), ep['W2'], ep['b2'].reshape(1, HD),
            ep['W3'], ep['b3'].reshape(1, HD))


def _split_upd_w(up):
    w1 = up['W1']
    return (w1[:HD], w1[HD:2 * HD], w1[2 * HD:], up['b1'].reshape(1, HD),
            up['W2'], up['b2'].reshape(1, HD))


def kernel(atom_num, dis1, dis2, id1u, id1v, id2u, id2v, params):
    p = params
    atom2d = atom_num.astype(jnp.int32).reshape(N, 1)
    dis1_2d = dis1.reshape(E, 1)
    dis2_2d = dis2.reshape(E, 1)
    i1u = id1u.astype(jnp.int32)
    i1v = id1v.astype(jnp.int32)
    i2u = id2u.astype(jnp.int32)
    i2v = id2v.astype(jnp.int32)
    i1v3 = i1v.reshape(-1, NMICRO, KMIC2)
    i2v3 = i2v.reshape(-1, NMICRO, KMIC2)
    z = jnp.zeros((N, HD), jnp.float32)

    emb = p['atom_emb']
    emb_pad = jnp.pad(emb, ((0, HD - emb.shape[0]), (0, 0)))

    e1u, e1v, e1d, e1b1, e1w2, e1b2, e1w3, e1b3 = _split_edge_w(p['edge1'])
    e2u, e2v, e2d, e2b1, e2w2, e2b2, e2w3, e2b3 = _split_edge_w(p['edge2'])
    f1u, f1v, f1d, f1b1, f1w2, f1b2, f1w3, f1b3 = _split_edge_w(p['uedge1'])
    f2u, f2v, f2d, f2b1, f2w2, f2b2, f2w3, f2b3 = _split_edge_w(p['uedge2'])
    u1h, u1a, u1b, u1b1, u1w2, u1b2 = _split_upd_w(p['upd1'])
    u2h, u2a, u2b, u2b1, u2w2, u2b2 = _split_upd_w(p['upd2'])

    wout_pad = jnp.pad(p['Wout'], ((0, 0), (0, 8 - p['Wout'].shape[1])))
    bout_pad = jnp.pad(p['bout'], (0, 8 - p['bout'].shape[0])).reshape(1, 8)

    h, pu1, pv1, pu2, pv2 = _tc_prep(atom2d, emb_pad, e1u, e1v, e1b1,
                                     e2u, e2v, e2b1)
    g1 = _sc_gather(pu1, pv1, i1u, i1v)
    g2 = _sc_gather(pu2, pv2, i2u, i2v)
    m1 = _tc_emlp(g1, dis1_2d, e1d, e1w2, e1b2, e1w3, e1b3)
    m2 = _tc_emlp(g2, dis2_2d, e2d, e2w2, e2b2, e2w3, e2b3)
    s1 = _sc_scatter(m1, i1v3, z)
    s2 = _sc_scatter(m2, i2v3, z)
    h1, qu1, qv1, qu2, qv2 = _tc_upd1(h, s1, s2, u1h, u1a, u1b, u1b1, u1w2,
                                      u1b2, f1u, f1v, f1b1, f2u, f2v, f2b1)

    g1 = _sc_gather(qu1, qv1, i1u, i1v)
    g2 = _sc_gather(qu2, qv2, i2u, i2v)
    m1 = _tc_emlp(g1, dis1_2d, f1d, f1w2, f1b2, f1w3, f1b3)
    m2 = _tc_emlp(g2, dis2_2d, f2d, f2w2, f2b2, f2w3, f2b3)
    s1 = _sc_scatter(m1, i1v3, z)
    s2 = _sc_scatter(m2, i2v3, z)
    delta8 = _tc_upd2(h1, s1, s2, u2h, u2a, u2b, u2b1, u2w2, u2b2,
                      wout_pad, bout_pad)
    return delta8[:, :3]


# speedup vs baseline: 1.1664x; 1.0431x over previous
import functools

import jax
import jax.numpy as jnp
from jax import lax
from jax.experimental import pallas as pl
from jax.experimental.pallas import tpu as pltpu
from jax.experimental.pallas import tpu_sc as plsc

HD = 128
N = 10000
E = 320000
LANES = 16

NB = 1000
BE = 2000
KMIC = 40
NMICRO = 5
KMAC = KMIC * NMICRO
KMIC2 = 80
NC = 2
NS = 16
NW = NC * NS
EPW = E // NW
NMAC = EPW // KMAC
NMACB = EPW // (KMIC2 * NMICRO)
ROWS_PT = 624
ROWS_LAST = N - (NS - 1) * ROWS_PT


def _silu(x):
    return x * lax.logistic(x)


def _prep_body(an, emb, w1u, w1v, b1, w2u, w2v, b2,
               h_o, pu1_o, pv1_o, pu2_o, pv2_o):
    iota = lax.broadcasted_iota(jnp.int32, (1, HD), 1)
    oh = (an[...] == iota).astype(jnp.float32)
    h = jnp.dot(oh, emb[...], preferred_element_type=jnp.float32)
    h_o[...] = h
    pu1_o[...] = jnp.dot(h, w1u[...], preferred_element_type=jnp.float32) + b1[...]
    pv1_o[...] = jnp.dot(h, w1v[...], preferred_element_type=jnp.float32)
    pu2_o[...] = jnp.dot(h, w2u[...], preferred_element_type=jnp.float32) + b2[...]
    pv2_o[...] = jnp.dot(h, w2v[...], preferred_element_type=jnp.float32)


def _tc_prep(atom2d, emb_pad, w1u, w1v, b1, w2u, w2v, b2):
    row = pl.BlockSpec((NB, HD), lambda i: (i, 0))
    wsp = pl.BlockSpec((HD, HD), lambda i: (0, 0))
    bsp = pl.BlockSpec((1, HD), lambda i: (0, 0))
    return pl.pallas_call(
        _prep_body,
        grid=(N // NB,),
        in_specs=[pl.BlockSpec((NB, 1), lambda i: (i, 0)),
                  wsp, wsp, wsp, bsp, wsp, wsp, bsp],
        out_specs=[row] * 5,
        out_shape=[jax.ShapeDtypeStruct((N, HD), jnp.float32)] * 5,
    )(atom2d, emb_pad, w1u, w1v, b1, w2u, w2v, b2)


def _emlp_body(g, dis, w1d, w2, b2, w3, b3, m_o):
    x = _silu(g[...] + dis[...] * w1d[...])
    x = _silu(jnp.dot(x, w2[...], preferred_element_type=jnp.float32) + b2[...])
    m_o[...] = jnp.dot(x, w3[...], preferred_element_type=jnp.float32) + b3[...]


def _tc_emlp(g, dis2d, w1d, w2, b2, w3, b3):
    row = pl.BlockSpec((BE, HD), lambda i: (i, 0))
    wsp = pl.BlockSpec((HD, HD), lambda i: (0, 0))
    bsp = pl.BlockSpec((1, HD), lambda i: (0, 0))
    return pl.pallas_call(
        _emlp_body,
        grid=(E // BE,),
        in_specs=[row, pl.BlockSpec((BE, 1), lambda i: (i, 0)),
                  bsp, wsp, bsp, wsp, bsp],
        out_specs=row,
        out_shape=jax.ShapeDtypeStruct((E, HD), jnp.float32),
    )(g, dis2d, w1d, w2, b2, w3, b3)


def _upd_common(h, s1, s2, uh, ua, ub, b1, w2, b2):
    a1 = s1[0] + s1[1]
    a2 = s2[0] + s2[1]
    x = _silu(jnp.dot(h[...], uh[...], preferred_element_type=jnp.float32)
              + jnp.dot(a1, ua[...], preferred_element_type=jnp.float32)
              + jnp.dot(a2, ub[...], preferred_element_type=jnp.float32)
              + b1[...])
    x = jnp.dot(x, w2[...], preferred_element_type=jnp.float32) + b2[...]
    return h[...] + x


def _upd1_body(h, s1, s2, uh, ua, ub, b1, w2, b2,
               qw1u, qw1v, qb1, qw2u, qw2v, qb2,
               h_o, pu1_o, pv1_o, pu2_o, pv2_o):
    hn = _upd_common(h, s1, s2, uh, ua, ub, b1, w2, b2)
    h_o[...] = hn
    pu1_o[...] = jnp.dot(hn, qw1u[...], preferred_element_type=jnp.float32) + qb1[...]
    pv1_o[...] = jnp.dot(hn, qw1v[...], preferred_element_type=jnp.float32)
    pu2_o[...] = jnp.dot(hn, qw2u[...], preferred_element_type=jnp.float32) + qb2[...]
    pv2_o[...] = jnp.dot(hn, qw2v[...], preferred_element_type=jnp.float32)


def _tc_upd1(h, s1, s2, uh, ua, ub, b1, w2, b2,
             qw1u, qw1v, qb1, qw2u, qw2v, qb2):
    row = pl.BlockSpec((NB, HD), lambda i: (i, 0))
    ssp = pl.BlockSpec((NC, NB, HD), lambda i: (0, i, 0))
    wsp = pl.BlockSpec((HD, HD), lambda i: (0, 0))
    bsp = pl.BlockSpec((1, HD), lambda i: (0, 0))
    return pl.pallas_call(
        _upd1_body,
        grid=(N // NB,),
        in_specs=[row, ssp, ssp, wsp, wsp, wsp, bsp, wsp, bsp,
                  wsp, wsp, bsp, wsp, wsp, bsp],
        out_specs=[row] * 5,
        out_shape=[jax.ShapeDtypeStruct((N, HD), jnp.float32)] * 5,
    )(h, s1, s2, uh, ua, ub, b1, w2, b2, qw1u, qw1v, qb1, qw2u, qw2v, qb2)


def _upd2_body(h, s1, s2, uh, ua, ub, b1, w2, b2, wout, bout, d_o):
    hn = _upd_common(h, s1, s2, uh, ua, ub, b1, w2, b2)
    d_o[...] = jnp.dot(hn, wout[...], preferred_element_type=jnp.float32) + bout[...]


def _tc_upd2(h, s1, s2, uh, ua, ub, b1, w2, b2, wout_pad, bout_pad):
    row = pl.BlockSpec((NB, HD), lambda i: (i, 0))
    ssp = pl.BlockSpec((NC, NB, HD), lambda i: (0, i, 0))
    wsp = pl.BlockSpec((HD, HD), lambda i: (0, 0))
    bsp = pl.BlockSpec((1, HD), lambda i: (0, 0))
    return pl.pallas_call(
        _upd2_body,
        grid=(N // NB,),
        in_specs=[row, ssp, ssp, wsp, wsp, wsp, bsp, wsp, bsp,
                  pl.BlockSpec((HD, 8), lambda i: (0, 0)),
                  pl.BlockSpec((1, 8), lambda i: (0, 0))],
        out_specs=pl.BlockSpec((NB, 8), lambda i: (i, 0)),
        out_shape=jax.ShapeDtypeStruct((N, 8), jnp.float32),
    )(h, s1, s2, uh, ua, ub, b1, w2, b2, wout_pad, bout_pad)


@functools.lru_cache(maxsize=None)
def _sc_gather_fn():
    mesh = plsc.VectorSubcoreMesh(core_axis_name="c", subcore_axis_name="s")

    @functools.partial(
        pl.kernel, mesh=mesh,
        out_type=jax.ShapeDtypeStruct((E, HD), jnp.float32),
        scratch_types=[
            pltpu.VMEM((EPW,), jnp.int32),
            pltpu.VMEM((EPW,), jnp.int32),
            pltpu.VMEM((KMAC, HD), jnp.float32),
            pltpu.VMEM((KMAC, HD), jnp.float32),
            pltpu.VMEM((KMAC, HD), jnp.float32),
            pltpu.VMEM((KMAC, HD), jnp.float32),
            pltpu.SemaphoreType.DMA,
            pltpu.SemaphoreType.DMA,
            pltpu.SemaphoreType.DMA,
        ])
    def _gather(pu_hbm, pv_hbm, iu_hbm, iv_hbm, out_hbm,
                iu_v, iv_v, a0_v, b0_v, a1_v, b1_v, sem0, sem1, sem_o):
        wid = lax.axis_index("s") * NC + lax.axis_index("c")
        ebase = wid * EPW
        pltpu.sync_copy(iu_hbm.at[pl.ds(ebase, EPW)], iu_v)
        pltpu.sync_copy(iv_hbm.at[pl.ds(ebase, EPW)], iv_v)
        slots = ((a0_v, b0_v, sem0), (a1_v, b1_v, sem1))

        def fire(sl, l0):
            a_v, b_v, sem = slots[sl]
            cps = []
            for j in range(NMICRO):
                src = pl.ds(l0 + sl * KMAC + j * KMIC, KMIC)
                dst = pl.ds(j * KMIC, KMIC)
                cps.append(pltpu.async_copy(pu_hbm.at[iu_v.at[src]],
                                            a_v.at[dst], sem))
                cps.append(pltpu.async_copy(pv_hbm.at[iv_v.at[src]],
                                            b_v.at[dst], sem))
            return cps

        def add_and_out(sl, l0):
            a_v, b_v, _ = slots[sl]

            def add_row(e, c2):
                for j in range(HD // LANES):
                    lsl = pl.ds(j * LANES, LANES)
                    a_v[e, lsl] = a_v[e, lsl] + b_v[e, lsl]
                return c2

            lax.fori_loop(0, KMAC, add_row, 0)
            return pltpu.async_copy(
                a_v, out_hbm.at[pl.ds(ebase + l0 + sl * KMAC, KMAC)], sem_o)

        def pair(i, carry):
            l0 = i * (2 * KMAC)
            cps0 = fire(0, l0)
            cps1 = fire(1, l0)
            for cp in cps0:
                cp.wait()
            o0 = add_and_out(0, l0)
            for cp in cps1:
                cp.wait()
            o1 = add_and_out(1, l0)
            o0.wait()
            o1.wait()
            return carry

        lax.fori_loop(0, NMAC // 2, pair, 0)

    return _gather


def _sc_gather(pu, pv, iu, iv):
    return _sc_gather_fn()(pu, pv, iu, iv)


@functools.lru_cache(maxsize=None)
def _sc_scatter_fn():
    mesh = plsc.VectorSubcoreMesh(core_axis_name="c", subcore_axis_name="s")

    @functools.partial(
        pl.kernel, mesh=mesh,
        out_type=jax.ShapeDtypeStruct((NC, N, HD), jnp.float32),
        scratch_types=[
            pltpu.VMEM((NMACB, NMICRO, KMIC2), jnp.int32),
            pltpu.VMEM((KMIC2, HD), jnp.float32),
            pltpu.VMEM((KMIC2, HD), jnp.float32),
            pltpu.VMEM_SHARED((N, HD), jnp.float32),
            pltpu.SemaphoreType.DMA,
            pltpu.SemaphoreType.DMA,
        ])
    def _scatter(m_hbm, iv_hbm, z_hbm, out_hbm, iv_v, m0_v, m1_v, s_sh,
                 sem_m, sem_s):
        c = lax.axis_index("c")
        s = lax.axis_index("s")
        wid = s * NC + c
        r0 = s * ROWS_PT

        @pl.when(s < NS - 1)
        def _():
            pltpu.sync_copy(z_hbm.at[pl.ds(r0, ROWS_PT)], s_sh.at[pl.ds(r0, ROWS_PT)])

        @pl.when(s == NS - 1)
        def _():
            pltpu.sync_copy(z_hbm.at[pl.ds(r0, ROWS_LAST)], s_sh.at[pl.ds(r0, ROWS_LAST)])

        plsc.subcore_barrier()
        ebase = wid * EPW

        bufs = (m0_v, m1_v)
        pltpu.sync_copy(iv_hbm.at[pl.ds(wid * NMACB, NMACB)], iv_v)

        def mac(ci, carry):
            e0 = ebase + ci * KMIC2 * NMICRO

            def mcp(j):
                return pltpu.async_copy(
                    m_hbm.at[pl.ds(e0 + j * KMIC2, KMIC2)],
                    bufs[j % 2], sem_m)

            def scp(j):
                return pltpu.async_copy(bufs[j % 2], s_sh.at[iv_v.at[ci, j]],
                                        sem_s, add=True)

            cpm = {0: mcp(0), 1: mcp(1)}
            cps = {}
            for j in range(NMICRO):
                cpm[j].wait()
                cps[j] = scp(j)
                if j + 2 < NMICRO:
                    cps[j].wait()
                    cpm[j + 2] = mcp(j + 2)
            for j in range(max(0, NMICRO - 2), NMICRO):
                cps[j].wait()
            return carry

        lax.fori_loop(0, NMACB, mac, 0)
        plsc.subcore_barrier()

        @pl.when(s < NS - 1)
        def _():
            pltpu.sync_copy(s_sh.at[pl.ds(r0, ROWS_PT)],
                            out_hbm.at[c].at[pl.ds(r0, ROWS_PT)])

        @pl.when(s == NS - 1)
        def _():
            pltpu.sync_copy(s_sh.at[pl.ds(r0, ROWS_LAST)],
                            out_hbm.at[c].at[pl.ds(r0, ROWS_LAST)])

    return _scatter


def _sc_scatter(m, iv, z):
    return _sc_scatter_fn()(m, iv, z)


def _split_edge_w(ep):
    w1 = ep['W1']
    return (w1[:HD], w1[HD:2 * HD], w1[2 * HD:2 * HD + 1],
            ep['b1'].reshape(1, HD), ep['W2'], ep['b2'].reshape(1, HD),
            ep['W3'], ep['b3'].reshape(1, HD))


def _split_upd_w(up):
    w1 = up['W1']
    return (w1[:HD], w1[HD:2 * HD], w1[2 * HD:], up['b1'].reshape(1, HD),
            up['W2'], up['b2'].reshape(1, HD))


def kernel(atom_num, dis1, dis2, id1u, id1v, id2u, id2v, params):
    p = params
    atom2d = atom_num.astype(jnp.int32).reshape(N, 1)
    dis1_2d = dis1.reshape(E, 1)
    dis2_2d = dis2.reshape(E, 1)
    i1u = id1u.astype(jnp.int32)
    i1v = id1v.astype(jnp.int32)
    i2u = id2u.astype(jnp.int32)
    i2v = id2v.astype(jnp.int32)
    i1v3 = i1v.reshape(-1, NMICRO, KMIC2)
    i2v3 = i2v.reshape(-1, NMICRO, KMIC2)
    z = jnp.zeros((N, HD), jnp.float32)

    emb = p['atom_emb']
    emb_pad = jnp.pad(emb, ((0, HD - emb.shape[0]), (0, 0)))

    e1u, e1v, e1d, e1b1, e1w2, e1b2, e1w3, e1b3 = _split_edge_w(p['edge1'])
    e2u, e2v, e2d, e2b1, e2w2, e2b2, e2w3, e2b3 = _split_edge_w(p['edge2'])
    f1u, f1v, f1d, f1b1, f1w2, f1b2, f1w3, f1b3 = _split_edge_w(p['uedge1'])
    f2u, f2v, f2d, f2b1, f2w2, f2b2, f2w3, f2b3 = _split_edge_w(p['uedge2'])
    u1h, u1a, u1b, u1b1, u1w2, u1b2 = _split_upd_w(p['upd1'])
    u2h, u2a, u2b, u2b1, u2w2, u2b2 = _split_upd_w(p['upd2'])

    wout_pad = jnp.pad(p['Wout'], ((0, 0), (0, 8 - p['Wout'].shape[1])))
    bout_pad = jnp.pad(p['bout'], (0, 8 - p['bout'].shape[0])).reshape(1, 8)

    h, pu1, pv1, pu2, pv2 = _tc_prep(atom2d, emb_pad, e1u, e1v, e1b1,
                                     e2u, e2v, e2b1)
    g1 = _sc_gather(pu1, pv1, i1u, i1v)
    g2 = _sc_gather(pu2, pv2, i2u, i2v)
    m1 = _tc_emlp(g1, dis1_2d, e1d, e1w2, e1b2, e1w3, e1b3)
    m2 = _tc_emlp(g2, dis2_2d, e2d, e2w2, e2b2, e2w3, e2b3)
    s1 = _sc_scatter(m1, i1v3, z)
    s2 = _sc_scatter(m2, i2v3, z)
    h1, qu1, qv1, qu2, qv2 = _tc_upd1(h, s1, s2, u1h, u1a, u1b, u1b1, u1w2,
                                      u1b2, f1u, f1v, f1b1, f2u, f2v, f2b1)

    g1 = _sc_gather(qu1, qv1, i1u, i1v)
    g2 = _sc_gather(qu2, qv2, i2u, i2v)
    m1 = _tc_emlp(g1, dis1_2d, f1d, f1w2, f1b2, f1w3, f1b3)
    m2 = _tc_emlp(g2, dis2_2d, f2d, f2w2, f2b2, f2w3, f2b3)
    s1 = _sc_scatter(m1, i1v3, z)
    s2 = _sc_scatter(m2, i2v3, z)
    delta8 = _tc_upd2(h1, s1, s2, u2h, u2a, u2b, u2b1, u2w2, u2b2,
                      wout_pad, bout_pad)
    return delta8[:, :3]


# speedup vs baseline: 1.2278x; 1.0526x over previous
import functools

import jax
import jax.numpy as jnp
from jax import lax
from jax.experimental import pallas as pl
from jax.experimental.pallas import tpu as pltpu
from jax.experimental.pallas import tpu_sc as plsc

HD = 128
N = 10000
E = 320000
E1 = 192000
LANES = 16

NB = 1000
BE = 2000
KMIC = 40
NMICRO = 5
KMAC = KMIC * NMICRO
KMIC2 = 80
NC = 2
NS = 16
NW = NC * NS
ROWS_PT = 624
ROWS_LAST = N - (NS - 1) * ROWS_PT


def _silu(x):
    return x * lax.logistic(x)


def _prep_body(an, emb, w1u, w1v, b1, w2u, w2v, b2,
               h_o, pu1_o, pv1_o, pu2_o, pv2_o):
    iota = lax.broadcasted_iota(jnp.int32, (1, HD), 1)
    oh = (an[...] == iota).astype(jnp.float32)
    h = jnp.dot(oh, emb[...], preferred_element_type=jnp.float32)
    h_o[...] = h
    pu1_o[...] = jnp.dot(h, w1u[...], preferred_element_type=jnp.float32) + b1[...]
    pv1_o[...] = jnp.dot(h, w1v[...], preferred_element_type=jnp.float32)
    pu2_o[...] = jnp.dot(h, w2u[...], preferred_element_type=jnp.float32) + b2[...]
    pv2_o[...] = jnp.dot(h, w2v[...], preferred_element_type=jnp.float32)


def _tc_prep(atom2d, emb_pad, w1u, w1v, b1, w2u, w2v, b2):
    row = pl.BlockSpec((NB, HD), lambda i: (i, 0))
    wsp = pl.BlockSpec((HD, HD), lambda i: (0, 0))
    bsp = pl.BlockSpec((1, HD), lambda i: (0, 0))
    return pl.pallas_call(
        _prep_body,
        grid=(N // NB,),
        in_specs=[pl.BlockSpec((NB, 1), lambda i: (i, 0)),
                  wsp, wsp, wsp, bsp, wsp, wsp, bsp],
        out_specs=[row] * 5,
        out_shape=[jax.ShapeDtypeStruct((N, HD), jnp.float32)] * 5,
    )(atom2d, emb_pad, w1u, w1v, b1, w2u, w2v, b2)


def _emlp_body(g, dis, w1d, w2, b2, w3, b3, m_o):
    x = _silu(g[...] + dis[...] * w1d[...])
    x = _silu(jnp.dot(x, w2[...], preferred_element_type=jnp.float32) + b2[...])
    m_o[...] = jnp.dot(x, w3[...], preferred_element_type=jnp.float32) + b3[...]


def _tc_emlp(g, dis2d, w1d, w2, b2, w3, b3):
    ee = g.shape[0]
    row = pl.BlockSpec((BE, HD), lambda i: (i, 0))
    wsp = pl.BlockSpec((HD, HD), lambda i: (0, 0))
    bsp = pl.BlockSpec((1, HD), lambda i: (0, 0))
    return pl.pallas_call(
        _emlp_body,
        grid=(ee // BE,),
        in_specs=[row, pl.BlockSpec((BE, 1), lambda i: (i, 0)),
                  bsp, wsp, bsp, wsp, bsp],
        out_specs=row,
        out_shape=jax.ShapeDtypeStruct((ee, HD), jnp.float32),
    )(g, dis2d, w1d, w2, b2, w3, b3)


def _upd_common(h, s1a, s1b, s2a, s2b, uh, ua, ub, b1, w2, b2):
    a1 = s1a[0] + s1a[1] + s1b[0] + s1b[1]
    a2 = s2a[0] + s2a[1] + s2b[0] + s2b[1]
    x = _silu(jnp.dot(h[...], uh[...], preferred_element_type=jnp.float32)
              + jnp.dot(a1, ua[...], preferred_element_type=jnp.float32)
              + jnp.dot(a2, ub[...], preferred_element_type=jnp.float32)
              + b1[...])
    x = jnp.dot(x, w2[...], preferred_element_type=jnp.float32) + b2[...]
    return h[...] + x


def _upd1_body(h, s1a, s1b, s2a, s2b, uh, ua, ub, b1, w2, b2,
               qw1u, qw1v, qb1, qw2u, qw2v, qb2,
               h_o, pu1_o, pv1_o, pu2_o, pv2_o):
    hn = _upd_common(h, s1a, s1b, s2a, s2b, uh, ua, ub, b1, w2, b2)
    h_o[...] = hn
    pu1_o[...] = jnp.dot(hn, qw1u[...], preferred_element_type=jnp.float32) + qb1[...]
    pv1_o[...] = jnp.dot(hn, qw1v[...], preferred_element_type=jnp.float32)
    pu2_o[...] = jnp.dot(hn, qw2u[...], preferred_element_type=jnp.float32) + qb2[...]
    pv2_o[...] = jnp.dot(hn, qw2v[...], preferred_element_type=jnp.float32)


def _tc_upd1(h, s1a, s1b, s2a, s2b, uh, ua, ub, b1, w2, b2,
             qw1u, qw1v, qb1, qw2u, qw2v, qb2):
    row = pl.BlockSpec((NB, HD), lambda i: (i, 0))
    ssp = pl.BlockSpec((NC, NB, HD), lambda i: (0, i, 0))
    wsp = pl.BlockSpec((HD, HD), lambda i: (0, 0))
    bsp = pl.BlockSpec((1, HD), lambda i: (0, 0))
    return pl.pallas_call(
        _upd1_body,
        grid=(N // NB,),
        in_specs=[row, ssp, ssp, ssp, ssp, wsp, wsp, wsp, bsp, wsp, bsp,
                  wsp, wsp, bsp, wsp, wsp, bsp],
        out_specs=[row] * 5,
        out_shape=[jax.ShapeDtypeStruct((N, HD), jnp.float32)] * 5,
    )(h, s1a, s1b, s2a, s2b, uh, ua, ub, b1, w2, b2,
      qw1u, qw1v, qb1, qw2u, qw2v, qb2)


def _upd2_body(h, s1a, s1b, s2a, s2b, uh, ua, ub, b1, w2, b2, wout, bout, d_o):
    hn = _upd_common(h, s1a, s1b, s2a, s2b, uh, ua, ub, b1, w2, b2)
    d_o[...] = jnp.dot(hn, wout[...], preferred_element_type=jnp.float32) + bout[...]


def _tc_upd2(h, s1a, s1b, s2a, s2b, uh, ua, ub, b1, w2, b2, wout_pad, bout_pad):
    row = pl.BlockSpec((NB, HD), lambda i: (i, 0))
    ssp = pl.BlockSpec((NC, NB, HD), lambda i: (0, i, 0))
    wsp = pl.BlockSpec((HD, HD), lambda i: (0, 0))
    bsp = pl.BlockSpec((1, HD), lambda i: (0, 0))
    return pl.pallas_call(
        _upd2_body,
        grid=(N // NB,),
        in_specs=[row, ssp, ssp, ssp, ssp, wsp, wsp, wsp, bsp, wsp, bsp,
                  pl.BlockSpec((HD, 8), lambda i: (0, 0)),
                  pl.BlockSpec((1, 8), lambda i: (0, 0))],
        out_specs=pl.BlockSpec((NB, 8), lambda i: (i, 0)),
        out_shape=jax.ShapeDtypeStruct((N, 8), jnp.float32),
    )(h, s1a, s1b, s2a, s2b, uh, ua, ub, b1, w2, b2, wout_pad, bout_pad)


@functools.lru_cache(maxsize=None)
def _sc_gather_fn(epw):
    ee = epw * NW
    nmac = epw // KMAC
    mesh = plsc.VectorSubcoreMesh(core_axis_name="c", subcore_axis_name="s")

    @functools.partial(
        pl.kernel, mesh=mesh,
        out_type=jax.ShapeDtypeStruct((ee, HD), jnp.float32),
        scratch_types=[
            pltpu.VMEM((epw,), jnp.int32),
            pltpu.VMEM((epw,), jnp.int32),
            pltpu.VMEM((KMAC, HD), jnp.float32),
            pltpu.VMEM((KMAC, HD), jnp.float32),
            pltpu.VMEM((KMAC, HD), jnp.float32),
            pltpu.VMEM((KMAC, HD), jnp.float32),
            pltpu.SemaphoreType.DMA,
            pltpu.SemaphoreType.DMA,
            pltpu.SemaphoreType.DMA,
        ])
    def _gather(pu_hbm, pv_hbm, iu_hbm, iv_hbm, out_hbm,
                iu_v, iv_v, a0_v, b0_v, a1_v, b1_v, sem0, sem1, sem_o):
        wid = lax.axis_index("s") * NC + lax.axis_index("c")
        ebase = wid * epw
        pltpu.sync_copy(iu_hbm.at[pl.ds(ebase, epw)], iu_v)
        pltpu.sync_copy(iv_hbm.at[pl.ds(ebase, epw)], iv_v)
        slots = ((a0_v, b0_v, sem0), (a1_v, b1_v, sem1))

        def fire(sl, l0):
            a_v, b_v, sem = slots[sl]
            cps = []
            for j in range(NMICRO):
                src = pl.ds(l0 + sl * KMAC + j * KMIC, KMIC)
                dst = pl.ds(j * KMIC, KMIC)
                cps.append(pltpu.async_copy(pu_hbm.at[iu_v.at[src]],
                                            a_v.at[dst], sem))
                cps.append(pltpu.async_copy(pv_hbm.at[iv_v.at[src]],
                                            b_v.at[dst], sem))
            return cps

        def add_and_out(sl, l0):
            a_v, b_v, _ = slots[sl]

            def add_row(e, c2):
                for j in range(HD // LANES):
                    lsl = pl.ds(j * LANES, LANES)
                    a_v[e, lsl] = a_v[e, lsl] + b_v[e, lsl]
                return c2

            lax.fori_loop(0, KMAC, add_row, 0)
            return pltpu.async_copy(
                a_v, out_hbm.at[pl.ds(ebase + l0 + sl * KMAC, KMAC)], sem_o)

        def pair(i, carry):
            l0 = i * (2 * KMAC)
            cps0 = fire(0, l0)
            cps1 = fire(1, l0)
            for cp in cps0:
                cp.wait()
            o0 = add_and_out(0, l0)
            for cp in cps1:
                cp.wait()
            o1 = add_and_out(1, l0)
            o0.wait()
            o1.wait()
            return carry

        lax.fori_loop(0, nmac // 2, pair, 0)

    return _gather


def _sc_gather(pu, pv, iu, iv):
    return _sc_gather_fn(iu.shape[0] // NW)(pu, pv, iu, iv)


@functools.lru_cache(maxsize=None)
def _sc_scatter_fn(epw):
    nmacb = epw // (KMIC2 * NMICRO)
    mesh = plsc.VectorSubcoreMesh(core_axis_name="c", subcore_axis_name="s")

    @functools.partial(
        pl.kernel, mesh=mesh,
        out_type=jax.ShapeDtypeStruct((NC, N, HD), jnp.float32),
        scratch_types=[
            pltpu.VMEM((nmacb, NMICRO, KMIC2), jnp.int32),
            pltpu.VMEM((KMIC2, HD), jnp.float32),
            pltpu.VMEM((KMIC2, HD), jnp.float32),
            pltpu.VMEM_SHARED((N, HD), jnp.float32),
            pltpu.SemaphoreType.DMA,
            pltpu.SemaphoreType.DMA,
        ])
    def _scatter(m_hbm, iv_hbm, z_hbm, out_hbm, iv_v, m0_v, m1_v, s_sh,
                 sem_m, sem_s):
        c = lax.axis_index("c")
        s = lax.axis_index("s")
        wid = s * NC + c
        r0 = s * ROWS_PT

        @pl.when(s < NS - 1)
        def _():
            pltpu.sync_copy(z_hbm.at[pl.ds(r0, ROWS_PT)], s_sh.at[pl.ds(r0, ROWS_PT)])

        @pl.when(s == NS - 1)
        def _():
            pltpu.sync_copy(z_hbm.at[pl.ds(r0, ROWS_LAST)], s_sh.at[pl.ds(r0, ROWS_LAST)])

        plsc.subcore_barrier()
        ebase = wid * epw
        bufs = (m0_v, m1_v)
        pltpu.sync_copy(iv_hbm.at[pl.ds(wid * nmacb, nmacb)], iv_v)

        def mac(ci, carry):
            e0 = ebase + ci * KMIC2 * NMICRO

            def mcp(j):
                return pltpu.async_copy(
                    m_hbm.at[pl.ds(e0 + j * KMIC2, KMIC2)],
                    bufs[j % 2], sem_m)

            def scp(j):
                return pltpu.async_copy(bufs[j % 2], s_sh.at[iv_v.at[ci, j]],
                                        sem_s, add=True)

            cpm = {0: mcp(0), 1: mcp(1)}
            cps = {}
            for j in range(NMICRO):
                cpm[j].wait()
                cps[j] = scp(j)
                if j + 2 < NMICRO:
                    cps[j].wait()
                    cpm[j + 2] = mcp(j + 2)
            for j in range(max(0, NMICRO - 2), NMICRO):
                cps[j].wait()
            return carry

        lax.fori_loop(0, nmacb, mac, 0)
        plsc.subcore_barrier()

        @pl.when(s < NS - 1)
        def _():
            pltpu.sync_copy(s_sh.at[pl.ds(r0, ROWS_PT)],
                            out_hbm.at[c].at[pl.ds(r0, ROWS_PT)])

        @pl.when(s == NS - 1)
        def _():
            pltpu.sync_copy(s_sh.at[pl.ds(r0, ROWS_LAST)],
                            out_hbm.at[c].at[pl.ds(r0, ROWS_LAST)])

    return _scatter


def _sc_scatter(m, iv3, z):
    return _sc_scatter_fn(m.shape[0] // NW)(m, iv3, z)


def _split_edge_w(ep):
    w1 = ep['W1']
    return (w1[:HD], w1[HD:2 * HD], w1[2 * HD:2 * HD + 1],
            ep['b1'].reshape(1, HD), ep['W2'], ep['b2'].reshape(1, HD),
            ep['W3'], ep['b3'].reshape(1, HD))


def _split_upd_w(up):
    w1 = up['W1']
    return (w1[:HD], w1[HD:2 * HD], w1[2 * HD:], up['b1'].reshape(1, HD),
            up['W2'], up['b2'].reshape(1, HD))


def _edge_layer(pu, pv, iu_h, iv_h, iv3_h, dis_h, ew, z):
    w1d, w2, b2, w3, b3 = ew
    outs = []
    for iu, iv, iv3, dis in zip(iu_h, iv_h, iv3_h, dis_h):
        g = _sc_gather(pu, pv, iu, iv)
        m = _tc_emlp(g, dis, w1d, w2, b2, w3, b3)
        outs.append(_sc_scatter(m, iv3, z))
    return outs


def kernel(atom_num, dis1, dis2, id1u, id1v, id2u, id2v, params):
    p = params
    atom2d = atom_num.astype(jnp.int32).reshape(N, 1)
    z = jnp.zeros((N, HD), jnp.float32)

    def split_e(x):
        return (x[:E1], x[E1:])

    def prep_idx(iu, iv):
        iu_h = split_e(iu.astype(jnp.int32))
        iv_h = split_e(iv.astype(jnp.int32))
        iv3_h = tuple(v.reshape(-1, NMICRO, KMIC2) for v in iv_h)
        return iu_h, iv_h, iv3_h

    i1u_h, i1v_h, i1v3_h = prep_idx(id1u, id1v)
    i2u_h, i2v_h, i2v3_h = prep_idx(id2u, id2v)
    dis1_h = tuple(v.reshape(-1, 1) for v in split_e(dis1))
    dis2_h = tuple(v.reshape(-1, 1) for v in split_e(dis2))

    emb = p['atom_emb']
    emb_pad = jnp.pad(emb, ((0, HD - emb.shape[0]), (0, 0)))

    e1u, e1v, e1d, e1b1, e1w2, e1b2, e1w3, e1b3 = _split_edge_w(p['edge1'])
    e2u, e2v, e2d, e2b1, e2w2, e2b2, e2w3, e2b3 = _split_edge_w(p['edge2'])
    f1u, f1v, f1d, f1b1, f1w2, f1b2, f1w3, f1b3 = _split_edge_w(p['uedge1'])
    f2u, f2v, f2d, f2b1, f2w2, f2b2, f2w3, f2b3 = _split_edge_w(p['uedge2'])
    u1h, u1a, u1b, u1b1, u1w2, u1b2 = _split_upd_w(p['upd1'])
    u2h, u2a, u2b, u2b1, u2w2, u2b2 = _split_upd_w(p['upd2'])

    wout_pad = jnp.pad(p['Wout'], ((0, 0), (0, 8 - p['Wout'].shape[1])))
    bout_pad = jnp.pad(p['bout'], (0, 8 - p['bout'].shape[0])).reshape(1, 8)

    h, pu1, pv1, pu2, pv2 = _tc_prep(atom2d, emb_pad, e1u, e1v, e1b1,
                                     e2u, e2v, e2b1)
    s1a, s1b = _edge_layer(pu1, pv1, i1u_h, i1v_h, i1v3_h, dis1_h,
                           (e1d, e1w2, e1b2, e1w3, e1b3), z)
    s2a, s2b = _edge_layer(pu2, pv2, i2u_h, i2v_h, i2v3_h, dis2_h,
                           (e2d, e2w2, e2b2, e2w3, e2b3), z)
    h1, qu1, qv1, qu2, qv2 = _tc_upd1(h, s1a, s1b, s2a, s2b,
                                      u1h, u1a, u1b, u1b1, u1w2, u1b2,
                                      f1u, f1v, f1b1, f2u, f2v, f2b1)

    s1a, s1b = _edge_layer(qu1, qv1, i1u_h, i1v_h, i1v3_h, dis1_h,
                           (f1d, f1w2, f1b2, f1w3, f1b3), z)
    s2a, s2b = _edge_layer(qu2, qv2, i2u_h, i2v_h, i2v3_h, dis2_h,
                           (f2d, f2w2, f2b2, f2w3, f2b3), z)
    delta8 = _tc_upd2(h1, s1a, s1b, s2a, s2b,
                      u2h, u2a, u2b, u2b1, u2w2, u2b2, wout_pad, bout_pad)
    return delta8[:, :3]
